# baseline (speedup 1.0000x reference)
import dataclasses
import functools

import jax
import jax.numpy as jnp
from jax import lax
from jax.experimental import pallas as pl
from jax.experimental.pallas import tpu as pltpu
from jax.experimental.pallas import tpu_sc as plsc

NUM_CORES = 2
NUM_SUBCORES = 16
NUM_TILES = NUM_CORES * NUM_SUBCORES
LANES = 16
D = 32
CHUNK = 128
SCORE_BLK = 32768
SC_COLS = 327680
SC_CHUNK = 512
SC_PER_TILE = SC_COLS // NUM_TILES
SC_BLOCKS = SC_COLS // SCORE_BLK


def _scores_body(ut_ref, mt_ref, wu_ref, wm_ref, us_ref, ms_ref):
    us_ref[...] = jnp.sum(ut_ref[...] * wu_ref[...], axis=0)
    ms_ref[...] = jnp.sum(mt_ref[...] * wm_ref[...], axis=0)


@functools.lru_cache(maxsize=None)
def _build_tc_scores(d: int, n_cols: int, c_tc: int, blk: int):
    grid = pl.cdiv(c_tc, blk)
    off = SC_BLOCKS
    return pl.pallas_call(
        _scores_body,
        grid=(grid,),
        in_specs=[
            pl.BlockSpec((d, blk), lambda j: (0, j + off)),
            pl.BlockSpec((d, blk), lambda j: (0, j + off)),
            pl.BlockSpec((d, 1), lambda j: (0, 0)),
            pl.BlockSpec((d, 1), lambda j: (0, 0)),
        ],
        out_specs=[
            pl.BlockSpec((blk,), lambda j: (j,)),
            pl.BlockSpec((blk,), lambda j: (j,)),
        ],
        out_shape=[jax.ShapeDtypeStruct((c_tc,), jnp.float32)] * 2,
        compiler_params=pltpu.CompilerParams(
            dimension_semantics=("parallel",)),
    )


def _sc_compiler_params():
    cp = pltpu.CompilerParams()
    if "needs_layout_passes" in pltpu.CompilerParams.__dataclass_fields__:
        cp = dataclasses.replace(cp, needs_layout_passes=False)
    return cp


@functools.lru_cache(maxsize=None)
def _build_sc_scores(d: int, sc_cols: int):
    assert sc_cols == SC_COLS and SC_PER_TILE % SC_CHUNK == 0
    n_chunks = SC_PER_TILE // SC_CHUNK
    mesh = plsc.VectorSubcoreMesh(core_axis_name="c", subcore_axis_name="s")

    @functools.partial(
        pl.kernel,
        out_type=[jax.ShapeDtypeStruct((sc_cols,), jnp.float32)] * 2,
        mesh=mesh,
        compiler_params=_sc_compiler_params(),
        scratch_types=[
            pltpu.VMEM((d, SC_CHUNK), jnp.float32),
            pltpu.VMEM((d, SC_CHUNK), jnp.float32),
            pltpu.VMEM((SC_CHUNK,), jnp.float32),
            pltpu.VMEM((SC_CHUNK,), jnp.float32),
            pltpu.VMEM((d * LANES,), jnp.float32),
            pltpu.VMEM((d * LANES,), jnp.float32),
            pltpu.SemaphoreType.DMA,
            pltpu.SemaphoreType.DMA,
        ],
    )
    def sc_scores(ut_h, mt_h, ubt_h, mbt_h, us_h, ms_h,
                  ubuf, mbuf, ustage, mstage, ubt, mbt, sem_u, sem_m):
        wid = lax.axis_index("s") * NUM_CORES + lax.axis_index("c")
        col0 = wid * SC_PER_TILE
        out0 = col0

        pltpu.sync_copy(ubt_h, ubt)
        pltpu.sync_copy(mbt_h, mbt)

        @pl.loop(0, n_chunks)
        def _(c):
            src = pl.ds(col0 + c * SC_CHUNK, SC_CHUNK)
            cu = pltpu.async_copy(ut_h.at[:, src], ubuf, sem_u)
            cm = pltpu.async_copy(mt_h.at[:, src], mbuf, sem_m)
            cu.wait()
            cm.wait()
            for k in range(d):
                wku = ubt[pl.ds(k * LANES, LANES)]
                wkm = mbt[pl.ds(k * LANES, LANES)]
                if k == 0:
                    @pl.loop(0, SC_CHUNK, step=LANES)
                    def _(j):
                        sl = pl.ds(j, LANES)
                        ustage[sl] = ubuf[0, sl] * wku
                        mstage[sl] = mbuf[0, sl] * wkm
                else:
                    @pl.loop(0, SC_CHUNK, step=LANES)
                    def _(j):
                        sl = pl.ds(j, LANES)
                        ustage[sl] = ustage[sl] + ubuf[k, sl] * wku
                        mstage[sl] = mstage[sl] + mbuf[k, sl] * wkm
            dst = pl.ds(out0 + c * SC_CHUNK, SC_CHUNK)
            pltpu.sync_copy(ustage, us_h.at[dst])
            pltpu.sync_copy(mstage, ms_h.at[dst])

    return sc_scores


@functools.lru_cache(maxsize=None)
def _build_gather(batch: int, c_tc: int, sc_cols: int):
    assert batch % (8 * NUM_TILES) == 0
    bpw = batch // NUM_TILES
    n_chunks = bpw // CHUNK
    mesh = plsc.VectorSubcoreMesh(core_axis_name="c", subcore_axis_name="s")

    @functools.partial(
        pl.kernel,
        out_type=jax.ShapeDtypeStruct((batch,), jnp.float32),
        mesh=mesh,
        compiler_params=_sc_compiler_params(),
        scratch_types=[
            pltpu.VMEM((bpw,), jnp.int32),
            pltpu.VMEM((bpw,), jnp.int32),
            pltpu.VMEM((bpw,), jnp.int32),
            pltpu.VMEM((bpw,), jnp.int32),
            pltpu.VMEM((bpw,), jnp.int32),
            pltpu.VMEM((bpw,), jnp.int32),
            pltpu.VMEM((bpw,), jnp.float32),
            pltpu.VMEM((bpw,), jnp.float32),
            pltpu.VMEM((bpw,), jnp.float32),
            pltpu.VMEM((bpw,), jnp.float32),
            pltpu.VMEM((bpw,), jnp.float32),
            pltpu.VMEM((LANES,), jnp.float32),
            pltpu.SemaphoreType.DMA,
            pltpu.SemaphoreType.DMA,
        ],
    )
    def gather_add(users_h, movies_h, ust_h, ussc_h, mst_h, mssc_h, bv_h,
                   out_h, uidx, midx, uitc, uisc, mitc, misc,
                   ustc, ussc, mstc, mssc, outv, bvv, sem_u, sem_m):
        wid = lax.axis_index("s") * NUM_CORES + lax.axis_index("c")
        base = wid * bpw

        pltpu.sync_copy(users_h.at[pl.ds(base, bpw)], uidx)
        pltpu.sync_copy(movies_h.at[pl.ds(base, bpw)], midx)
        pltpu.sync_copy(bv_h, bvv)

        iota = lax.iota(jnp.int32, LANES)

        @pl.loop(0, bpw, step=LANES)
        def _(i):
            sl = pl.ds(i, LANES)
            spread = i + iota
            u = uidx[sl]
            m = midx[sl]
            uitc[sl] = jnp.where(u < sc_cols, spread, u - sc_cols)
            uisc[sl] = jnp.where(u < sc_cols, u, spread)
            mitc[sl] = jnp.where(m < sc_cols, spread, m - sc_cols)
            misc[sl] = jnp.where(m < sc_cols, m, spread)

        copies = []
        for c in range(n_chunks):
            sl = pl.ds(c * CHUNK, CHUNK)
            copies.append(
                pltpu.async_copy(ust_h.at[uitc.at[sl]], ustc.at[sl], sem_u))
            copies.append(
                pltpu.async_copy(ussc_h.at[uisc.at[sl]], ussc.at[sl], sem_u))
            copies.append(
                pltpu.async_copy(mst_h.at[mitc.at[sl]], mstc.at[sl], sem_m))
            copies.append(
                pltpu.async_copy(mssc_h.at[misc.at[sl]], mssc.at[sl], sem_m))
        for cp_ in copies:
            cp_.wait()

        bvec = bvv[...]

        @pl.loop(0, bpw, step=LANES)
        def _(i):
            sl = pl.ds(i, LANES)
            us = jnp.where(uidx[sl] < sc_cols, ussc[sl], ustc[sl])
            ms = jnp.where(midx[sl] < sc_cols, mssc[sl], mstc[sl])
            outv[sl] = us + ms + bvec

        pltpu.sync_copy(outv, out_h.at[pl.ds(base, bpw)])

    return gather_add


def kernel(users, movies, user_table, movie_table, W, b):
    batch = users.shape[0]
    n_rows, d = user_table.shape
    c_tc = n_rows - SC_COLS
    users = users.astype(jnp.int32)
    movies = movies.astype(jnp.int32)
    utT = user_table.T
    mtT = movie_table.T
    wu = W[0, :d].reshape(d, 1).astype(jnp.float32)
    wm = W[0, d:].reshape(d, 1).astype(jnp.float32)
    ubt = jnp.repeat(W[0, :d].astype(jnp.float32), LANES)
    mbt = jnp.repeat(W[0, d:].astype(jnp.float32), LANES)
    us_tc, ms_tc = _build_tc_scores(d, n_rows, c_tc, SCORE_BLK)(
        utT, mtT, wu, wm)
    us_sc, ms_sc = _build_sc_scores(d, SC_COLS)(utT, mtT, ubt, mbt)
    bv = jnp.broadcast_to(b, (LANES,)).astype(jnp.float32)
    out = _build_gather(batch, c_tc, SC_COLS)(
        users, movies, us_tc, us_sc, ms_tc, ms_sc, bv)
    return out.reshape(batch, 1)


# speedup vs baseline: 2.7316x; 2.7316x over previous
import dataclasses
import functools

import jax
import jax.numpy as jnp
from jax import lax
from jax.experimental import pallas as pl
from jax.experimental.pallas import tpu as pltpu
from jax.experimental.pallas import tpu_sc as plsc

NUM_CORES = 2
NUM_SUBCORES = 16
NUM_TILES = NUM_CORES * NUM_SUBCORES
LANES = 16
D = 32
CHUNK = 128
SCORE_BLK = 32768
SC_COLS = 327680
SC_CHUNK = 512
SC_PER_TILE = SC_COLS // NUM_TILES
SC_BLOCKS = SC_COLS // SCORE_BLK


def _scores_body(ut_ref, mt_ref, wu_ref, wm_ref, us_ref, ms_ref):
    us_ref[...] = jnp.sum(ut_ref[...] * wu_ref[...], axis=0)
    ms_ref[...] = jnp.sum(mt_ref[...] * wm_ref[...], axis=0)


@functools.lru_cache(maxsize=None)
def _build_tc_scores(d: int, n_cols: int, c_tc: int, blk: int):
    grid = pl.cdiv(c_tc, blk)
    off = SC_BLOCKS
    return pl.pallas_call(
        _scores_body,
        grid=(grid,),
        in_specs=[
            pl.BlockSpec((d, blk), lambda j: (0, j + off)),
            pl.BlockSpec((d, blk), lambda j: (0, j + off)),
            pl.BlockSpec((d, 1), lambda j: (0, 0)),
            pl.BlockSpec((d, 1), lambda j: (0, 0)),
        ],
        out_specs=[
            pl.BlockSpec((blk,), lambda j: (j,)),
            pl.BlockSpec((blk,), lambda j: (j,)),
        ],
        out_shape=[jax.ShapeDtypeStruct((c_tc,), jnp.float32)] * 2,
        compiler_params=pltpu.CompilerParams(
            dimension_semantics=("parallel",)),
    )


def _sc_compiler_params():
    cp = pltpu.CompilerParams()
    if "needs_layout_passes" in pltpu.CompilerParams.__dataclass_fields__:
        cp = dataclasses.replace(cp, needs_layout_passes=False)
    return cp


@functools.lru_cache(maxsize=None)
def _build_sc_scores(d: int, sc_cols: int):
    assert sc_cols == SC_COLS and SC_PER_TILE % SC_CHUNK == 0
    n_chunks = SC_PER_TILE // SC_CHUNK
    mesh = plsc.VectorSubcoreMesh(core_axis_name="c", subcore_axis_name="s")

    @functools.partial(
        pl.kernel,
        out_type=[jax.ShapeDtypeStruct((sc_cols,), jnp.float32)] * 2,
        mesh=mesh,
        compiler_params=_sc_compiler_params(),
        scratch_types=[
            pltpu.VMEM((d, SC_CHUNK), jnp.float32),
            pltpu.VMEM((d, SC_CHUNK), jnp.float32),
            pltpu.VMEM((d, SC_CHUNK), jnp.float32),
            pltpu.VMEM((d, SC_CHUNK), jnp.float32),
            pltpu.VMEM((SC_PER_TILE,), jnp.float32),
            pltpu.VMEM((SC_PER_TILE,), jnp.float32),
            pltpu.VMEM((d * LANES,), jnp.float32),
            pltpu.VMEM((d * LANES,), jnp.float32),
            pltpu.SemaphoreType.DMA,
            pltpu.SemaphoreType.DMA,
        ],
    )
    def sc_scores(ut_h, mt_h, ubt_h, mbt_h, us_h, ms_h,
                  ubufa, ubufb, mbufa, mbufb, uout, mout, ubt, mbt,
                  sem_a, sem_b):
        wid = lax.axis_index("s") * NUM_CORES + lax.axis_index("c")
        col0 = wid * SC_PER_TILE

        pltpu.sync_copy(ubt_h, ubt)
        pltpu.sync_copy(mbt_h, mbt)

        def start(c, ub, mb, sem):
            src = pl.ds(col0 + c * SC_CHUNK, SC_CHUNK)
            pltpu.async_copy(ut_h.at[:, src], ub, sem)
            pltpu.async_copy(mt_h.at[:, src], mb, sem)

        def drain(c, ub, mb, sem):
            src = pl.ds(col0 + c * SC_CHUNK, SC_CHUNK)
            pltpu.make_async_copy(ut_h.at[:, src], ub, sem).wait()
            pltpu.make_async_copy(mt_h.at[:, src], mb, sem).wait()

        nq = 4

        def compute(c, ub, mb):
            for buf, out, wref in ((ub, uout, ubt), (mb, mout, mbt)):
                @pl.loop(0, SC_CHUNK, step=nq * LANES)
                def _(j):
                    sls = [pl.ds(j + q * LANES, LANES) for q in range(nq)]
                    wk = wref[pl.ds(0, LANES)]
                    acc = [buf[0, s] * wk for s in sls]
                    for k in range(1, d):
                        wk = wref[pl.ds(k * LANES, LANES)]
                        acc = [a + buf[k, s] * wk
                               for a, s in zip(acc, sls)]
                    base = c * SC_CHUNK + j
                    for q, a in enumerate(acc):
                        out[pl.ds(base + q * LANES, LANES)] = a

        start(0, ubufa, mbufa, sem_a)

        @pl.loop(0, n_chunks, step=2)
        def _(c):
            start(c + 1, ubufb, mbufb, sem_b)
            drain(c, ubufa, mbufa, sem_a)
            compute(c, ubufa, mbufa)

            @pl.when(c + 2 < n_chunks)
            def _():
                start(c + 2, ubufa, mbufa, sem_a)

            drain(c + 1, ubufb, mbufb, sem_b)
            compute(c + 1, ubufb, mbufb)

        dst = pl.ds(col0, SC_PER_TILE)
        pltpu.sync_copy(uout, us_h.at[dst])
        pltpu.sync_copy(mout, ms_h.at[dst])

    return sc_scores


@functools.lru_cache(maxsize=None)
def _build_gather(batch: int, c_tc: int, sc_cols: int):
    assert batch % (8 * NUM_TILES) == 0
    bpw = batch // NUM_TILES
    n_chunks = bpw // CHUNK
    mesh = plsc.VectorSubcoreMesh(core_axis_name="c", subcore_axis_name="s")

    @functools.partial(
        pl.kernel,
        out_type=jax.ShapeDtypeStruct((batch,), jnp.float32),
        mesh=mesh,
        compiler_params=_sc_compiler_params(),
        scratch_types=[
            pltpu.VMEM((bpw,), jnp.int32),
            pltpu.VMEM((bpw,), jnp.int32),
            pltpu.VMEM((bpw,), jnp.int32),
            pltpu.VMEM((bpw,), jnp.int32),
            pltpu.VMEM((bpw,), jnp.int32),
            pltpu.VMEM((bpw,), jnp.int32),
            pltpu.VMEM((bpw,), jnp.float32),
            pltpu.VMEM((bpw,), jnp.float32),
            pltpu.VMEM((bpw,), jnp.float32),
            pltpu.VMEM((bpw,), jnp.float32),
            pltpu.VMEM((bpw,), jnp.float32),
            pltpu.VMEM((LANES,), jnp.float32),
            pltpu.SemaphoreType.DMA,
            pltpu.SemaphoreType.DMA,
        ],
    )
    def gather_add(users_h, movies_h, ust_h, ussc_h, mst_h, mssc_h, bv_h,
                   out_h, uidx, midx, uitc, uisc, mitc, misc,
                   ustc, ussc, mstc, mssc, outv, bvv, sem_u, sem_m):
        wid = lax.axis_index("s") * NUM_CORES + lax.axis_index("c")
        base = wid * bpw

        pltpu.sync_copy(users_h.at[pl.ds(base, bpw)], uidx)
        pltpu.sync_copy(movies_h.at[pl.ds(base, bpw)], midx)
        pltpu.sync_copy(bv_h, bvv)

        iota = lax.iota(jnp.int32, LANES)

        @pl.loop(0, bpw, step=LANES)
        def _(i):
            sl = pl.ds(i, LANES)
            spread = i + iota
            u = uidx[sl]
            m = midx[sl]
            uitc[sl] = jnp.where(u < sc_cols, spread, u - sc_cols)
            uisc[sl] = jnp.where(u < sc_cols, u, spread)
            mitc[sl] = jnp.where(m < sc_cols, spread, m - sc_cols)
            misc[sl] = jnp.where(m < sc_cols, m, spread)

        copies = []
        for c in range(n_chunks):
            sl = pl.ds(c * CHUNK, CHUNK)
            copies.append(
                pltpu.async_copy(ust_h.at[uitc.at[sl]], ustc.at[sl], sem_u))
            copies.append(
                pltpu.async_copy(ussc_h.at[uisc.at[sl]], ussc.at[sl], sem_u))
            copies.append(
                pltpu.async_copy(mst_h.at[mitc.at[sl]], mstc.at[sl], sem_m))
            copies.append(
                pltpu.async_copy(mssc_h.at[misc.at[sl]], mssc.at[sl], sem_m))
        for cp_ in copies:
            cp_.wait()

        bvec = bvv[...]

        @pl.loop(0, bpw, step=LANES)
        def _(i):
            sl = pl.ds(i, LANES)
            us = jnp.where(uidx[sl] < sc_cols, ussc[sl], ustc[sl])
            ms = jnp.where(midx[sl] < sc_cols, mssc[sl], mstc[sl])
            outv[sl] = us + ms + bvec

        pltpu.sync_copy(outv, out_h.at[pl.ds(base, bpw)])

    return gather_add


def kernel(users, movies, user_table, movie_table, W, b):
    batch = users.shape[0]
    n_rows, d = user_table.shape
    c_tc = n_rows - SC_COLS
    users = users.astype(jnp.int32)
    movies = movies.astype(jnp.int32)
    utT = user_table.T
    mtT = movie_table.T
    wu = W[0, :d].reshape(d, 1).astype(jnp.float32)
    wm = W[0, d:].reshape(d, 1).astype(jnp.float32)
    ubt = jnp.repeat(W[0, :d].astype(jnp.float32), LANES)
    mbt = jnp.repeat(W[0, d:].astype(jnp.float32), LANES)
    us_tc, ms_tc = _build_tc_scores(d, n_rows, c_tc, SCORE_BLK)(
        utT, mtT, wu, wm)
    us_sc, ms_sc = _build_sc_scores(d, SC_COLS)(utT, mtT, ubt, mbt)
    bv = jnp.broadcast_to(b, (LANES,)).astype(jnp.float32)
    out = _build_gather(batch, c_tc, SC_COLS)(
        users, movies, us_tc, us_sc, ms_tc, ms_sc, bv)
    return out.reshape(batch, 1)


# speedup vs baseline: 3.0099x; 1.1019x over previous
import dataclasses
import functools

import jax
import jax.numpy as jnp
from jax import lax
from jax.experimental import pallas as pl
from jax.experimental.pallas import tpu as pltpu
from jax.experimental.pallas import tpu_sc as plsc

NUM_CORES = 2
NUM_SUBCORES = 16
NUM_TILES = NUM_CORES * NUM_SUBCORES
LANES = 16
D = 32
CHUNK = 128
SCORE_BLK = 32768
SC_COLS = 327680
SC_CHUNK = 512
SC_PER_TILE = SC_COLS // NUM_TILES
SC_BLOCKS = SC_COLS // SCORE_BLK


def _scores_body(ut_ref, mt_ref, wu_ref, wm_ref, us_ref, ms_ref):
    us_ref[...] = jnp.sum(ut_ref[...] * wu_ref[...], axis=0)
    ms_ref[...] = jnp.sum(mt_ref[...] * wm_ref[...], axis=0)


@functools.lru_cache(maxsize=None)
def _build_tc_scores(d: int, n_cols: int, c_tc: int, blk: int):
    grid = pl.cdiv(c_tc, blk)
    off = SC_BLOCKS
    return pl.pallas_call(
        _scores_body,
        grid=(grid,),
        in_specs=[
            pl.BlockSpec((d, blk), lambda j: (0, j + off)),
            pl.BlockSpec((d, blk), lambda j: (0, j + off)),
            pl.BlockSpec((d, 1), lambda j: (0, 0)),
            pl.BlockSpec((d, 1), lambda j: (0, 0)),
        ],
        out_specs=[
            pl.BlockSpec((blk,), lambda j: (j,)),
            pl.BlockSpec((blk,), lambda j: (j,)),
        ],
        out_shape=[jax.ShapeDtypeStruct((c_tc,), jnp.float32)] * 2,
        compiler_params=pltpu.CompilerParams(
            dimension_semantics=("parallel",)),
    )


def _sc_compiler_params():
    cp = pltpu.CompilerParams()
    if "needs_layout_passes" in pltpu.CompilerParams.__dataclass_fields__:
        cp = dataclasses.replace(cp, needs_layout_passes=False)
    return cp


@functools.lru_cache(maxsize=None)
def _build_sc_scores(d: int, sc_cols: int):
    assert sc_cols == SC_COLS and SC_PER_TILE % SC_CHUNK == 0
    n_chunks = SC_PER_TILE // SC_CHUNK
    mesh = plsc.VectorSubcoreMesh(core_axis_name="c", subcore_axis_name="s")

    @functools.partial(
        pl.kernel,
        out_type=[jax.ShapeDtypeStruct((sc_cols,), jnp.float32)] * 2,
        mesh=mesh,
        compiler_params=_sc_compiler_params(),
        scratch_types=[
            pltpu.VMEM((d, SC_CHUNK), jnp.float32),
            pltpu.VMEM((d, SC_CHUNK), jnp.float32),
            pltpu.VMEM((d, SC_CHUNK), jnp.float32),
            pltpu.VMEM((d, SC_CHUNK), jnp.float32),
            pltpu.VMEM((SC_PER_TILE,), jnp.float32),
            pltpu.VMEM((SC_PER_TILE,), jnp.float32),
            pltpu.VMEM((d * LANES,), jnp.float32),
            pltpu.VMEM((d * LANES,), jnp.float32),
            pltpu.SemaphoreType.DMA,
            pltpu.SemaphoreType.DMA,
        ],
    )
    def sc_scores(ut_h, mt_h, ubt_h, mbt_h, us_h, ms_h,
                  ubufa, ubufb, mbufa, mbufb, uout, mout, ubt, mbt,
                  sem_a, sem_b):
        wid = lax.axis_index("s") * NUM_CORES + lax.axis_index("c")
        col0 = wid * SC_PER_TILE

        pltpu.sync_copy(ubt_h, ubt)
        pltpu.sync_copy(mbt_h, mbt)

        def start(c, ub, mb, sem):
            src = pl.ds(col0 + c * SC_CHUNK, SC_CHUNK)
            pltpu.async_copy(ut_h.at[:, src], ub, sem)
            pltpu.async_copy(mt_h.at[:, src], mb, sem)

        def drain(c, ub, mb, sem):
            src = pl.ds(col0 + c * SC_CHUNK, SC_CHUNK)
            pltpu.make_async_copy(ut_h.at[:, src], ub, sem).wait()
            pltpu.make_async_copy(mt_h.at[:, src], mb, sem).wait()

        nq = 4

        def compute(c, ub, mb):
            for buf, out, wref in ((ub, uout, ubt), (mb, mout, mbt)):
                @pl.loop(0, SC_CHUNK, step=nq * LANES)
                def _(j):
                    sls = [pl.ds(j + q * LANES, LANES) for q in range(nq)]
                    wk = wref[pl.ds(0, LANES)]
                    acc = [buf[0, s] * wk for s in sls]
                    for k in range(1, d):
                        wk = wref[pl.ds(k * LANES, LANES)]
                        acc = [a + buf[k, s] * wk
                               for a, s in zip(acc, sls)]
                    base = c * SC_CHUNK + j
                    for q, a in enumerate(acc):
                        out[pl.ds(base + q * LANES, LANES)] = a

        start(0, ubufa, mbufa, sem_a)

        @pl.loop(0, n_chunks, step=2)
        def _(c):
            start(c + 1, ubufb, mbufb, sem_b)
            drain(c, ubufa, mbufa, sem_a)
            compute(c, ubufa, mbufa)

            @pl.when(c + 2 < n_chunks)
            def _():
                start(c + 2, ubufa, mbufa, sem_a)

            drain(c + 1, ubufb, mbufb, sem_b)
            compute(c + 1, ubufb, mbufb)

        dst = pl.ds(col0, SC_PER_TILE)
        pltpu.sync_copy(uout, us_h.at[dst])
        pltpu.sync_copy(mout, ms_h.at[dst])

    return sc_scores


@functools.lru_cache(maxsize=None)
def _build_gather(batch: int):
    assert batch % (8 * NUM_TILES) == 0
    bpw = batch // NUM_TILES
    n_chunks = bpw // CHUNK
    mesh = plsc.VectorSubcoreMesh(core_axis_name="c", subcore_axis_name="s")

    @functools.partial(
        pl.kernel,
        out_type=jax.ShapeDtypeStruct((batch,), jnp.float32),
        mesh=mesh,
        compiler_params=_sc_compiler_params(),
        scratch_types=[
            pltpu.VMEM((bpw,), jnp.int32),
            pltpu.VMEM((bpw,), jnp.int32),
            pltpu.VMEM((bpw,), jnp.float32),
            pltpu.VMEM((bpw,), jnp.float32),
            pltpu.VMEM((bpw,), jnp.float32),
            pltpu.VMEM((LANES,), jnp.float32),
            pltpu.SemaphoreType.DMA,
            pltpu.SemaphoreType.DMA,
        ],
    )
    def gather_add(users_h, movies_h, us_h, ms_h, bv_h, out_h,
                   uidx, midx, usv, msv, outv, bvv, sem_u, sem_m):
        wid = lax.axis_index("s") * NUM_CORES + lax.axis_index("c")
        base = wid * bpw

        pltpu.sync_copy(users_h.at[pl.ds(base, bpw)], uidx)
        pltpu.sync_copy(movies_h.at[pl.ds(base, bpw)], midx)
        pltpu.sync_copy(bv_h, bvv)

        copies = []
        for c in range(n_chunks):
            sl = pl.ds(c * CHUNK, CHUNK)
            copies.append(
                pltpu.async_copy(us_h.at[uidx.at[sl]], usv.at[sl], sem_u))
            copies.append(
                pltpu.async_copy(ms_h.at[midx.at[sl]], msv.at[sl], sem_m))
        for cp_ in copies:
            cp_.wait()

        bvec = bvv[...]

        @pl.loop(0, bpw, step=LANES)
        def _(i):
            sl = pl.ds(i, LANES)
            outv[sl] = usv[sl] + msv[sl] + bvec

        pltpu.sync_copy(outv, out_h.at[pl.ds(base, bpw)])

    return gather_add


def kernel(users, movies, user_table, movie_table, W, b):
    batch = users.shape[0]
    n_rows, d = user_table.shape
    c_tc = n_rows - SC_COLS
    users = users.astype(jnp.int32)
    movies = movies.astype(jnp.int32)
    utT = user_table.T
    mtT = movie_table.T
    wu = W[0, :d].reshape(d, 1).astype(jnp.float32)
    wm = W[0, d:].reshape(d, 1).astype(jnp.float32)
    ubt = jnp.repeat(W[0, :d].astype(jnp.float32), LANES)
    mbt = jnp.repeat(W[0, d:].astype(jnp.float32), LANES)
    us_tc, ms_tc = _build_tc_scores(d, n_rows, c_tc, SCORE_BLK)(
        utT, mtT, wu, wm)
    us_sc, ms_sc = _build_sc_scores(d, SC_COLS)(utT, mtT, ubt, mbt)
    us = jnp.concatenate([us_sc, us_tc])
    ms = jnp.concatenate([ms_sc, ms_tc])
    bv = jnp.broadcast_to(b, (LANES,)).astype(jnp.float32)
    out = _build_gather(batch)(users, movies, us, ms, bv)
    return out.reshape(batch, 1)


# speedup vs baseline: 3.0265x; 1.0055x over previous
import dataclasses
import functools

import jax
import jax.numpy as jnp
from jax import lax
from jax.experimental import pallas as pl
from jax.experimental.pallas import tpu as pltpu
from jax.experimental.pallas import tpu_sc as plsc

NUM_CORES = 2
NUM_SUBCORES = 16
NUM_TILES = NUM_CORES * NUM_SUBCORES
LANES = 16
D = 32
CHUNK = 128
SCORE_BLK = 32768
SC_COLS = 131072
SC_CHUNK = 512
SC_PER_TILE = SC_COLS // NUM_TILES
SC_BLOCKS = SC_COLS // SCORE_BLK


def _scores_body(ut_ref, mt_ref, wu_ref, wm_ref, us_ref, ms_ref):
    us_ref[...] = jnp.sum(ut_ref[...] * wu_ref[...], axis=0)
    ms_ref[...] = jnp.sum(mt_ref[...] * wm_ref[...], axis=0)


@functools.lru_cache(maxsize=None)
def _build_tc_scores(d: int, n_cols: int, c_tc: int, blk: int):
    grid = pl.cdiv(c_tc, blk)
    off = SC_BLOCKS
    return pl.pallas_call(
        _scores_body,
        grid=(grid,),
        in_specs=[
            pl.BlockSpec((d, blk), lambda j: (0, j + off)),
            pl.BlockSpec((d, blk), lambda j: (0, j + off)),
            pl.BlockSpec((d, 1), lambda j: (0, 0)),
            pl.BlockSpec((d, 1), lambda j: (0, 0)),
        ],
        out_specs=[
            pl.BlockSpec((blk,), lambda j: (j,)),
            pl.BlockSpec((blk,), lambda j: (j,)),
        ],
        out_shape=[jax.ShapeDtypeStruct((c_tc,), jnp.float32)] * 2,
        compiler_params=pltpu.CompilerParams(
            dimension_semantics=("parallel",)),
    )


def _sc_compiler_params():
    cp = pltpu.CompilerParams()
    if "needs_layout_passes" in pltpu.CompilerParams.__dataclass_fields__:
        cp = dataclasses.replace(cp, needs_layout_passes=False)
    return cp


@functools.lru_cache(maxsize=None)
def _build_sc_scores(d: int, sc_cols: int):
    assert sc_cols == SC_COLS and SC_PER_TILE % SC_CHUNK == 0
    n_chunks = SC_PER_TILE // SC_CHUNK
    mesh = plsc.VectorSubcoreMesh(core_axis_name="c", subcore_axis_name="s")

    @functools.partial(
        pl.kernel,
        out_type=[jax.ShapeDtypeStruct((sc_cols,), jnp.float32)] * 2,
        mesh=mesh,
        compiler_params=_sc_compiler_params(),
        scratch_types=[
            pltpu.VMEM((d, SC_CHUNK), jnp.float32),
            pltpu.VMEM((d, SC_CHUNK), jnp.float32),
            pltpu.VMEM((d, SC_CHUNK), jnp.float32),
            pltpu.VMEM((d, SC_CHUNK), jnp.float32),
            pltpu.VMEM((SC_PER_TILE,), jnp.float32),
            pltpu.VMEM((SC_PER_TILE,), jnp.float32),
            pltpu.VMEM((d * LANES,), jnp.float32),
            pltpu.VMEM((d * LANES,), jnp.float32),
            pltpu.SemaphoreType.DMA,
            pltpu.SemaphoreType.DMA,
        ],
    )
    def sc_scores(ut_h, mt_h, ubt_h, mbt_h, us_h, ms_h,
                  ubufa, ubufb, mbufa, mbufb, uout, mout, ubt, mbt,
                  sem_a, sem_b):
        wid = lax.axis_index("s") * NUM_CORES + lax.axis_index("c")
        col0 = wid * SC_PER_TILE

        pltpu.sync_copy(ubt_h, ubt)
        pltpu.sync_copy(mbt_h, mbt)

        def start(c, ub, mb, sem):
            src = pl.ds(col0 + c * SC_CHUNK, SC_CHUNK)
            pltpu.async_copy(ut_h.at[:, src], ub, sem)
            pltpu.async_copy(mt_h.at[:, src], mb, sem)

        def drain(c, ub, mb, sem):
            src = pl.ds(col0 + c * SC_CHUNK, SC_CHUNK)
            pltpu.make_async_copy(ut_h.at[:, src], ub, sem).wait()
            pltpu.make_async_copy(mt_h.at[:, src], mb, sem).wait()

        nq = 4

        def compute(c, ub, mb):
            for buf, out, wref in ((ub, uout, ubt), (mb, mout, mbt)):
                @pl.loop(0, SC_CHUNK, step=nq * LANES)
                def _(j):
                    sls = [pl.ds(j + q * LANES, LANES) for q in range(nq)]
                    wk = wref[pl.ds(0, LANES)]
                    acc = [buf[0, s] * wk for s in sls]
                    for k in range(1, d):
                        wk = wref[pl.ds(k * LANES, LANES)]
                        acc = [a + buf[k, s] * wk
                               for a, s in zip(acc, sls)]
                    base = c * SC_CHUNK + j
                    for q, a in enumerate(acc):
                        out[pl.ds(base + q * LANES, LANES)] = a

        start(0, ubufa, mbufa, sem_a)

        @pl.loop(0, n_chunks, step=2)
        def _(c):
            start(c + 1, ubufb, mbufb, sem_b)
            drain(c, ubufa, mbufa, sem_a)
            compute(c, ubufa, mbufa)

            @pl.when(c + 2 < n_chunks)
            def _():
                start(c + 2, ubufa, mbufa, sem_a)

            drain(c + 1, ubufb, mbufb, sem_b)
            compute(c + 1, ubufb, mbufb)

        dst = pl.ds(col0, SC_PER_TILE)
        pltpu.sync_copy(uout, us_h.at[dst])
        pltpu.sync_copy(mout, ms_h.at[dst])

    return sc_scores


@functools.lru_cache(maxsize=None)
def _build_gather(batch: int):
    assert batch % (8 * NUM_TILES) == 0
    bpw = batch // NUM_TILES
    n_chunks = bpw // CHUNK
    mesh = plsc.VectorSubcoreMesh(core_axis_name="c", subcore_axis_name="s")

    @functools.partial(
        pl.kernel,
        out_type=jax.ShapeDtypeStruct((batch,), jnp.float32),
        mesh=mesh,
        compiler_params=_sc_compiler_params(),
        scratch_types=[
            pltpu.VMEM((bpw,), jnp.int32),
            pltpu.VMEM((bpw,), jnp.int32),
            pltpu.VMEM((bpw,), jnp.float32),
            pltpu.VMEM((bpw,), jnp.float32),
            pltpu.VMEM((bpw,), jnp.float32),
            pltpu.VMEM((LANES,), jnp.float32),
            pltpu.SemaphoreType.DMA,
            pltpu.SemaphoreType.DMA,
        ],
    )
    def gather_add(users_h, movies_h, us_h, ms_h, bv_h, out_h,
                   uidx, midx, usv, msv, outv, bvv, sem_u, sem_m):
        wid = lax.axis_index("s") * NUM_CORES + lax.axis_index("c")
        base = wid * bpw

        pltpu.sync_copy(users_h.at[pl.ds(base, bpw)], uidx)
        pltpu.sync_copy(movies_h.at[pl.ds(base, bpw)], midx)
        pltpu.sync_copy(bv_h, bvv)

        copies = []
        for c in range(n_chunks):
            sl = pl.ds(c * CHUNK, CHUNK)
            copies.append(
                pltpu.async_copy(us_h.at[uidx.at[sl]], usv.at[sl], sem_u))
            copies.append(
                pltpu.async_copy(ms_h.at[midx.at[sl]], msv.at[sl], sem_m))
        for cp_ in copies:
            cp_.wait()

        bvec = bvv[...]

        @pl.loop(0, bpw, step=LANES)
        def _(i):
            sl = pl.ds(i, LANES)
            outv[sl] = usv[sl] + msv[sl] + bvec

        pltpu.sync_copy(outv, out_h.at[pl.ds(base, bpw)])

    return gather_add


def kernel(users, movies, user_table, movie_table, W, b):
    batch = users.shape[0]
    n_rows, d = user_table.shape
    c_tc = n_rows - SC_COLS
    users = users.astype(jnp.int32)
    movies = movies.astype(jnp.int32)
    utT = user_table.T
    mtT = movie_table.T
    wu = W[0, :d].reshape(d, 1).astype(jnp.float32)
    wm = W[0, d:].reshape(d, 1).astype(jnp.float32)
    ubt = jnp.repeat(W[0, :d].astype(jnp.float32), LANES)
    mbt = jnp.repeat(W[0, d:].astype(jnp.float32), LANES)
    us_tc, ms_tc = _build_tc_scores(d, n_rows, c_tc, SCORE_BLK)(
        utT, mtT, wu, wm)
    us_sc, ms_sc = _build_sc_scores(d, SC_COLS)(utT, mtT, ubt, mbt)
    us = jnp.concatenate([us_sc, us_tc])
    ms = jnp.concatenate([ms_sc, ms_tc])
    bv = jnp.broadcast_to(b, (LANES,)).astype(jnp.float32)
    out = _build_gather(batch)(users, movies, us, ms, bv)
    return out.reshape(batch, 1)


# speedup vs baseline: 3.3118x; 1.0943x over previous
import dataclasses
import functools

import jax
import jax.numpy as jnp
from jax import lax
from jax.experimental import pallas as pl
from jax.experimental.pallas import tpu as pltpu
from jax.experimental.pallas import tpu_sc as plsc

NUM_CORES = 2
NUM_SUBCORES = 16
NUM_TILES = NUM_CORES * NUM_SUBCORES
LANES = 16
D = 32
CHUNK = 128
SCORE_BLK = 32768


def _scores_body(ut_ref, mt_ref, wu_ref, wm_ref, us_ref, ms_ref):
    us_ref[...] = jnp.sum(ut_ref[...] * wu_ref[...], axis=0)
    ms_ref[...] = jnp.sum(mt_ref[...] * wm_ref[...], axis=0)


@functools.lru_cache(maxsize=None)
def _build_scores(n_rows: int, d: int, blk: int):
    grid = pl.cdiv(n_rows, blk)
    return pl.pallas_call(
        _scores_body,
        grid=(grid,),
        in_specs=[
            pl.BlockSpec((d, blk), lambda j: (0, j)),
            pl.BlockSpec((d, blk), lambda j: (0, j)),
            pl.BlockSpec((d, 1), lambda j: (0, 0)),
            pl.BlockSpec((d, 1), lambda j: (0, 0)),
        ],
        out_specs=[
            pl.BlockSpec((blk,), lambda j: (j,)),
            pl.BlockSpec((blk,), lambda j: (j,)),
        ],
        out_shape=[jax.ShapeDtypeStruct((n_rows,), jnp.float32)] * 2,
        compiler_params=pltpu.CompilerParams(
            dimension_semantics=("parallel",)),
    )


@functools.lru_cache(maxsize=None)
def _build_gather(batch: int):
    assert batch % (8 * NUM_TILES) == 0
    bpw = batch // NUM_TILES
    n_chunks = bpw // CHUNK

    mesh = plsc.VectorSubcoreMesh(core_axis_name="c", subcore_axis_name="s")
    cp = pltpu.CompilerParams()
    if "needs_layout_passes" in pltpu.CompilerParams.__dataclass_fields__:
        cp = dataclasses.replace(cp, needs_layout_passes=False)

    @functools.partial(
        pl.kernel,
        out_type=jax.ShapeDtypeStruct((batch,), jnp.float32),
        mesh=mesh,
        compiler_params=cp,
        scratch_types=[
            pltpu.VMEM((bpw,), jnp.int32),
            pltpu.VMEM((bpw,), jnp.int32),
            pltpu.VMEM((bpw,), jnp.float32),
            pltpu.VMEM((bpw,), jnp.float32),
            pltpu.VMEM((bpw,), jnp.float32),
            pltpu.VMEM((LANES,), jnp.float32),
            pltpu.SemaphoreType.DMA,
            pltpu.SemaphoreType.DMA,
        ],
    )
    def gather_add(users_h, movies_h, us_h, ms_h, bv_h, out_h,
                   uidx, midx, usv, msv, outv, bvv, sem_u, sem_m):
        wid = lax.axis_index("s") * NUM_CORES + lax.axis_index("c")
        base = wid * bpw

        pltpu.sync_copy(users_h.at[pl.ds(base, bpw)], uidx)
        pltpu.sync_copy(movies_h.at[pl.ds(base, bpw)], midx)
        pltpu.sync_copy(bv_h, bvv)

        copies = []
        for c in range(n_chunks):
            sl = pl.ds(c * CHUNK, CHUNK)
            copies.append(
                pltpu.async_copy(us_h.at[uidx.at[sl]], usv.at[sl], sem_u))
            copies.append(
                pltpu.async_copy(ms_h.at[midx.at[sl]], msv.at[sl], sem_m))
        for cp_ in copies:
            cp_.wait()

        bvec = bvv[...]

        @pl.loop(0, bpw, step=LANES)
        def _(i):
            sl = pl.ds(i, LANES)
            outv[sl] = usv[sl] + msv[sl] + bvec

        pltpu.sync_copy(outv, out_h.at[pl.ds(base, bpw)])

    return gather_add


def kernel(users, movies, user_table, movie_table, W, b):
    batch = users.shape[0]
    n_rows, d = user_table.shape
    users = users.astype(jnp.int32)
    movies = movies.astype(jnp.int32)
    utT = user_table.T
    mtT = movie_table.T
    wu = W[0, :d].reshape(d, 1).astype(jnp.float32)
    wm = W[0, d:].reshape(d, 1).astype(jnp.float32)
    uscore, mscore = _build_scores(n_rows, d, SCORE_BLK)(utT, mtT, wu, wm)
    bv = jnp.broadcast_to(b, (LANES,)).astype(jnp.float32)
    out = _build_gather(batch)(users, movies, uscore, mscore, bv)
    return out.reshape(batch, 1)
